# baseline (speedup 1.0000x reference)
import functools

import jax
import jax.numpy as jnp
from jax.experimental import pallas as pl
from jax.experimental.pallas import tpu as pltpu

B = 1
S = 2048
D_MODEL = 1024
D_FF = 4096
E = 8
TOPK = 2
C = 1024

BM = 512
BF = 512
NI = C // BM
NJ = D_FF // BF


def _ffn_body(buf_ref, w1_ref, w2_ref, out_ref, acc_ref):
    j = pl.program_id(2)

    @pl.when(j == 0)
    def _():
        acc_ref[...] = jnp.zeros_like(acc_ref)

    h = jnp.dot(buf_ref[0], w1_ref[0], preferred_element_type=jnp.float32)
    h = jax.nn.gelu(h)
    acc_ref[...] += jnp.dot(h, w2_ref[0], preferred_element_type=jnp.float32)

    @pl.when(j == NJ - 1)
    def _():
        out_ref[0] = acc_ref[...]


def _ffn(buf, w1, w2):
    return pl.pallas_call(
        _ffn_body,
        grid=(E, NI, NJ),
        in_specs=[
            pl.BlockSpec((1, BM, D_MODEL), lambda e, i, j: (e, i, 0)),
            pl.BlockSpec((1, D_MODEL, BF), lambda e, i, j: (e, 0, j)),
            pl.BlockSpec((1, BF, D_MODEL), lambda e, i, j: (e, j, 0)),
        ],
        out_specs=pl.BlockSpec((1, BM, D_MODEL), lambda e, i, j: (e, i, 0)),
        out_shape=jax.ShapeDtypeStruct((E, C, D_MODEL), jnp.float32),
        scratch_shapes=[pltpu.VMEM((BM, D_MODEL), jnp.float32)],
    )(buf, w1, w2)


def kernel(x, w_router, w1, w2):
    Bs, Ss, D = x.shape
    T = Bs * Ss
    xf = x.reshape(T, D)
    logits = xf @ w_router
    probs = jax.nn.softmax(logits, axis=-1)
    top_vals, top_idx = jax.lax.top_k(probs, TOPK)
    A = T * TOPK
    expert_id = top_idx.reshape(A)
    weight = top_vals.reshape(A)
    token_id = jnp.repeat(jnp.arange(T), TOPK)
    onehot = jax.nn.one_hot(expert_id, E, dtype=jnp.int32)
    pos = (jnp.cumsum(onehot, axis=0) * onehot).sum(-1) - 1
    keep = pos < C
    keepf = keep.astype(xf.dtype)
    disp = jnp.clip(expert_id * C + pos, 0, E * C - 1)
    buf = jnp.zeros((E * C, D), xf.dtype).at[disp].add(xf[token_id] * keepf[:, None])
    buf = buf.reshape(E, C, D)
    y = _ffn(buf, w1, w2).reshape(E * C, D)
    out = jnp.zeros((T, D), xf.dtype).at[token_id].add(
        weight[:, None] * keepf[:, None] * y[disp])
    return out.reshape(Bs, Ss, D)


# speedup vs baseline: 1.0713x; 1.0713x over previous
import functools

import jax
import jax.numpy as jnp
from jax.experimental import pallas as pl
from jax.experimental.pallas import tpu as pltpu

B = 1
S = 2048
D_MODEL = 1024
D_FF = 4096
E = 8
TOPK = 2
C = 1024
T = B * S
TB = 512


def _router_body(x_ref, wr_ref, slot0_ref, slot1_ref, keep0_ref, keep1_ref,
                 wt0_ref, wt1_ref, ccnt_ref, u_scr, p_scr):
    x = x_ref[...]
    wr = wr_ref[...]
    logits = jax.lax.dot_general(
        x, wr, (((1,), (0,)), ((), ())),
        preferred_element_type=jnp.float32)
    m = jnp.max(logits, axis=1, keepdims=True)
    ex = jnp.exp(logits - m)
    probs = ex / jnp.sum(ex, axis=1, keepdims=True)
    lane = jax.lax.broadcasted_iota(jnp.int32, (T, E), 1)
    m1 = jnp.max(probs, axis=1, keepdims=True)
    i1 = jnp.min(jnp.where(probs == m1, lane, E), axis=1, keepdims=True)
    probs2 = jnp.where(lane == i1, -1.0, probs)
    m2 = jnp.max(probs2, axis=1, keepdims=True)
    i2 = jnp.min(jnp.where(probs2 == m2, lane, E), axis=1, keepdims=True)
    u = (lane == i1).astype(jnp.float32) + (lane == i2).astype(jnp.float32)
    u_scr[...] = u
    rr = jax.lax.broadcasted_iota(jnp.int32, (TB, TB), 0)
    cc = jax.lax.broadcasted_iota(jnp.int32, (TB, TB), 1)
    tri = (cc < rr).astype(jnp.bfloat16)

    def blk(b, carry):
        ub = u_scr[pl.ds(b * TB, TB), :]
        pb = jax.lax.dot_general(
            tri, ub.astype(jnp.bfloat16), (((1,), (0,)), ((), ())),
            preferred_element_type=jnp.float32)
        p_scr[pl.ds(b * TB, TB), :] = pb + carry
        return carry + jnp.sum(ub, axis=0, keepdims=True)

    cnt = jax.lax.fori_loop(0, T // TB, blk, jnp.zeros((1, E), jnp.float32))
    p = p_scr[...]
    pos0 = jnp.sum(jnp.where(lane == i1, p, 0.0), axis=1, keepdims=True)
    pos1 = jnp.sum(jnp.where(lane == i2, p, 0.0), axis=1, keepdims=True)
    pos0i = pos0.astype(jnp.int32)
    pos1i = pos1.astype(jnp.int32)
    keep0 = pos0i < C
    keep1 = pos1i < C
    slot0_ref[...] = jnp.clip(i1 * C + pos0i, 0, E * C - 1)
    slot1_ref[...] = jnp.clip(i2 * C + pos1i, 0, E * C - 1)
    keep0_ref[...] = keep0.astype(jnp.int32)
    keep1_ref[...] = keep1.astype(jnp.int32)
    wt0_ref[...] = jnp.where(keep0, m1, 0.0)
    wt1_ref[...] = jnp.where(keep1, m2, 0.0)
    ccnt_ref[...] = jnp.minimum(cnt, float(C)).astype(jnp.int32)


def _router(xf, w_router):
    col_i = jax.ShapeDtypeStruct((T, 1), jnp.int32)
    col_f = jax.ShapeDtypeStruct((T, 1), jnp.float32)
    outs = pl.pallas_call(
        _router_body,
        in_specs=[
            pl.BlockSpec((T, D_MODEL), lambda: (0, 0)),
            pl.BlockSpec((D_MODEL, E), lambda: (0, 0)),
        ],
        out_specs=[pl.BlockSpec(o.shape, lambda: (0, 0)) for o in
                   (col_i, col_i, col_i, col_i, col_f, col_f,
                    jax.ShapeDtypeStruct((1, E), jnp.int32))],
        out_shape=[col_i, col_i, col_i, col_i, col_f, col_f,
                   jax.ShapeDtypeStruct((1, E), jnp.int32)],
        scratch_shapes=[pltpu.VMEM((T, E), jnp.float32),
                        pltpu.VMEM((T, E), jnp.float32)],
    )(xf, w_router)
    slot0, slot1, keep0, keep1, wt0, wt1, ccnt = outs
    return (slot0.reshape(T), slot1.reshape(T), keep0.reshape(T),
            keep1.reshape(T), wt0.reshape(T), wt1.reshape(T), ccnt.reshape(E))

BM = 512
BF = 512
NI = C // BM
NJ = D_FF // BF


def _ffn_body(buf_ref, w1_ref, w2_ref, out_ref, acc_ref):
    j = pl.program_id(2)

    @pl.when(j == 0)
    def _():
        acc_ref[...] = jnp.zeros_like(acc_ref)

    h = jnp.dot(buf_ref[0].astype(jnp.bfloat16), w1_ref[0].astype(jnp.bfloat16),
                preferred_element_type=jnp.float32)
    h = jax.nn.gelu(h)
    acc_ref[...] += jnp.dot(h.astype(jnp.bfloat16), w2_ref[0].astype(jnp.bfloat16),
                            preferred_element_type=jnp.float32)

    @pl.when(j == NJ - 1)
    def _():
        out_ref[0] = acc_ref[...]


def _ffn(buf, w1, w2):
    return pl.pallas_call(
        _ffn_body,
        grid=(E, NI, NJ),
        in_specs=[
            pl.BlockSpec((1, BM, D_MODEL), lambda e, i, j: (e, i, 0)),
            pl.BlockSpec((1, D_MODEL, BF), lambda e, i, j: (e, 0, j)),
            pl.BlockSpec((1, BF, D_MODEL), lambda e, i, j: (e, j, 0)),
        ],
        out_specs=pl.BlockSpec((1, BM, D_MODEL), lambda e, i, j: (e, i, 0)),
        out_shape=jax.ShapeDtypeStruct((E, C, D_MODEL), jnp.float32),
        scratch_shapes=[pltpu.VMEM((BM, D_MODEL), jnp.float32)],
    )(buf, w1, w2)


def kernel(x, w_router, w1, w2):
    Bs, Ss, D = x.shape
    xf = x.reshape(T, D)
    slot0, slot1, keep0, keep1, wt0, wt1, ccnt = _router(xf, w_router)
    k0 = keep0.astype(xf.dtype)[:, None]
    k1 = keep1.astype(xf.dtype)[:, None]
    buf = (jnp.zeros((E * C, D), xf.dtype)
           .at[slot0].add(xf * k0)
           .at[slot1].add(xf * k1))
    buf = buf.reshape(E, C, D)
    y = _ffn(buf, w1, w2).reshape(E * C, D)
    out = wt0[:, None] * y[slot0] + wt1[:, None] * y[slot1]
    return out.reshape(Bs, Ss, D)
